# Initial kernel scaffold; baseline (speedup 1.0000x reference)
import jax
import jax.numpy as jnp
from jax.experimental import pallas as pl


def kernel(pc2g_edge_idx, pc2g_edge_features, pc2g_node_features, g2g_edge_idx, g2g_edge_features, g2pc_edge_idx, g2pc_edge_features, params):
    raise NotImplementedError("write your pallas kernel here")


# speedup vs baseline: 2.6101x; 2.6101x over previous
import functools

import jax
import jax.numpy as jnp
from jax import lax
from jax.experimental import pallas as pl
from jax.experimental.pallas import tpu as pltpu
from jax.experimental.pallas import tpu_sc as plsc

F32 = jnp.float32
H = 128
_NG = 2500
NW = 32
CH = 80
EBLK = NW * CH


def _silu(x):
    return x * jax.nn.sigmoid(x)


def _ln(h, g, b):
    mu = jnp.mean(h, axis=-1, keepdims=True)
    var = jnp.mean((h - mu) ** 2, axis=-1, keepdims=True)
    return (h - mu) * lax.rsqrt(var + 1e-5) * g + b


def _pick_rows(n, cap=2048):
    best = None
    for cand in range(8, min(n, cap) + 1, 8):
        if n % cand == 0:
            best = cand
    return best if best is not None else n


def _mm(a, b):
    return jnp.dot(a, b, preferred_element_type=F32)


def _rowwise(fn, xs, ws, out_widths, R=None):
    E = xs[0].shape[0]
    if R is None:
        R = _pick_rows(E)
    grid = (E // R,)
    nx, nw = len(xs), len(ws)
    in_specs = [pl.BlockSpec((R, x.shape[1]), lambda i: (i, 0)) for x in xs]
    in_specs += [pl.BlockSpec(w.shape, lambda i: (0, 0)) for w in ws]
    out_specs = [pl.BlockSpec((R, wd), lambda i: (i, 0)) for wd in out_widths]
    out_shape = [jax.ShapeDtypeStruct((E, wd), F32) for wd in out_widths]

    def body(*refs):
        xr = [r[...] for r in refs[:nx]]
        wr = [r[...] for r in refs[nx:nx + nw]]
        outs = fn(*xr, *wr)
        if not isinstance(outs, tuple):
            outs = (outs,)
        for o_ref, o in zip(refs[nx + nw:], outs):
            o_ref[...] = o

    res = pl.pallas_call(body, grid=grid, in_specs=in_specs,
                         out_specs=out_specs, out_shape=out_shape)(*xs, *ws)
    return res


def _ffb_weights(p):
    ws = [p["W1"], p["b1"].reshape(1, -1), p["W2"], p["b2"].reshape(1, -1)]
    if "g" in p:
        ws += [p["g"].reshape(1, -1), p["beta"].reshape(1, -1)]
    return ws


def _ffb(x, p, act=_silu):
    has_ln = "g" in p

    def fn(xv, w1, b1, w2, b2, *rest):
        h = act(_mm(xv, w1) + b1)
        h = _mm(h, w2) + b2
        if has_ln:
            h = _ln(h, rest[0], rest[1])
        return h

    return _rowwise(fn, [x], _ffb_weights(p), [p["W2"].shape[1]])[0]


def _edge_weights(p):
    w1 = p["W1"]
    return [w1[:H], w1[H:2 * H], w1[2 * H:], p["b1"].reshape(1, -1),
            p["W2"], p["b2"].reshape(1, -1),
            p["g"].reshape(1, -1), p["beta"].reshape(1, -1)]


def _edge_mlp(ns, nr, ef, p):
    def fn(a, b, c, w1a, w1b, w1c, b1, w2, b2, g, beta):
        h = _silu(_mm(a, w1a) + _mm(b, w1b) + _mm(c, w1c) + b1)
        return _ln(_mm(h, w2) + b2, g, beta)

    return _rowwise(fn, [ns, nr, ef], _edge_weights(p), [H])[0]


def _proc_edge_mlp(ns, nr, pe, p):
    def fn(a, b, c, w1a, w1b, w1c, b1, w2, b2, g, beta):
        h = _silu(_mm(a, w1a) + _mm(b, w1b) + _mm(c, w1c) + b1)
        ne = _ln(_mm(h, w2) + b2, g, beta)
        return ne, c + ne

    return _rowwise(fn, [ns, nr, pe], _edge_weights(p), [H, H])


def _agg(p0, p1, c0, c1):
    cnt = jnp.maximum(c0[:, :1] + c1[:, :1], 1.0)
    return (p0 + p1) / cnt


def _node2_weights(p):
    w1 = p["W1"]
    return [w1[:H], w1[H:], p["b1"].reshape(1, -1), p["W2"],
            p["b2"].reshape(1, -1), p["g"].reshape(1, -1),
            p["beta"].reshape(1, -1)]


def _enc_node(n0, p0, p1, c0, c1, pn, po):
    def fn(nv, a0, a1, k0, k1, w1a, w1b, b1, w2, b2, g, beta,
           w1o, b1o, w2o, b2o):
        agg = _agg(a0, a1, k0, k1)
        h = _silu(_mm(nv, w1a) + _mm(agg, w1b) + b1)
        v = nv + _ln(_mm(h, w2) + b2, g, beta)
        h2 = _silu(_mm(v, w1o) + b1o)
        return _mm(h2, w2o) + b2o

    ws = _node2_weights(pn) + [po["W1"], po["b1"].reshape(1, -1),
                               po["W2"], po["b2"].reshape(1, -1)]
    return _rowwise(fn, [n0, p0, p1, c0, c1], ws, [H])[0]


def _proc_node(nl, p0, p1, c0, c1, pn):
    def fn(nv, a0, a1, k0, k1, w1a, w1b, b1, w2, b2, g, beta):
        agg = _agg(a0, a1, k0, k1)
        h = _silu(_mm(nv, w1a) + _mm(agg, w1b) + b1)
        return nv + _ln(_mm(h, w2) + b2, g, beta)

    return _rowwise(fn, [nl, p0, p1, c0, c1], _node2_weights(pn), [H])[0]


def _dec_node(n, p0, p1, c0, c1, pn, po):
    dout = po["W2"].shape[1]

    def fn(nv, a0, a1, k0, k1, w1a, w1b, b1, w2, b2, g, beta,
           w1o, b1o, w2o, b2o):
        agg = _agg(a0, a1, k0, k1)
        h = _silu(_mm(nv, w1a) + _mm(agg, w1b) + b1)
        nn2 = _ln(_mm(h, w2) + b2, g, beta)
        h2 = jax.nn.sigmoid(_mm(nn2, w1o) + b1o)
        return _mm(h2, w2o) + b2o

    ws = _node2_weights(pn) + [po["W1"], po["b1"].reshape(1, -1),
                               po["W2"], po["b2"].reshape(1, -1)]
    return _rowwise(fn, [n, p0, p1, c0, c1], ws, [dout])[0]


def _sc_mesh():
    return plsc.VectorSubcoreMesh(core_axis_name="c", subcore_axis_name="s")


def _sc_gather_pair(t1, i1, t2, i2):
    E = i1.shape[0]
    assert E % EBLK == 0, E
    rows = E // NW
    nch = rows // CH

    @functools.partial(
        pl.kernel, mesh=_sc_mesh(),
        out_type=(jax.ShapeDtypeStruct((E, H), F32),
                  jax.ShapeDtypeStruct((E, H), F32)),
        scratch_types=[pltpu.VMEM((CH,), jnp.int32),
                       pltpu.VMEM((CH,), jnp.int32),
                       pltpu.VMEM((CH, H), F32),
                       pltpu.VMEM((CH, H), F32),
                       pltpu.SemaphoreType.DMA,
                       pltpu.SemaphoreType.DMA],
    )
    def k(t1r, i1r, t2r, i2r, o1r, o2r, iv1, iv2, rb1, rb2, sm1, sm2):
        wid = lax.axis_index("s") * 2 + lax.axis_index("c")

        def body(ci, carry):
            base = wid * rows + ci * CH
            pltpu.sync_copy(i1r.at[pl.ds(base, CH)], iv1)
            pltpu.sync_copy(i2r.at[pl.ds(base, CH)], iv2)
            c1 = pltpu.async_copy(t1r.at[iv1], rb1, sm1)
            c2 = pltpu.async_copy(t2r.at[iv2], rb2, sm2)
            c1.wait()
            c2.wait()
            pltpu.sync_copy(rb1, o1r.at[pl.ds(base, CH)])
            pltpu.sync_copy(rb2, o2r.at[pl.ds(base, CH)])
            return carry

        lax.fori_loop(0, nch, body, 0)

    return k(t1, i1, t2, i2)


def _sc_scatter(vals, idx, npad):
    E = idx.shape[0]
    assert E % EBLK == 0 and npad % 128 == 0, (E, npad)
    rows = E // NW
    nch = rows // CH
    zr = npad // 16
    zrows = jnp.zeros((zr, H), F32)

    @functools.partial(
        pl.kernel, mesh=_sc_mesh(),
        out_type=jax.ShapeDtypeStruct((2, npad, H), F32),
        scratch_types=[pltpu.VMEM((CH,), jnp.int32),
                       pltpu.VMEM((CH, H), F32),
                       pltpu.VMEM_SHARED((npad, H), F32)],
    )
    def k(valsr, idxr, zrr, sumsr, iv, vb, acc):
        c = lax.axis_index("c")
        s = lax.axis_index("s")
        wid = s * 2 + c
        pltpu.sync_copy(zrr, acc.at[pl.ds(s * zr, zr)])
        plsc.subcore_barrier()

        def body(ci, carry):
            base = wid * rows + ci * CH
            pltpu.sync_copy(idxr.at[pl.ds(base, CH)], iv)
            pltpu.sync_copy(valsr.at[pl.ds(base, CH)], vb)
            pltpu.sync_copy(vb, acc.at[iv], add=True)
            return carry

        lax.fori_loop(0, nch, body, 0)
        plsc.subcore_barrier()
        pltpu.sync_copy(acc.at[pl.ds(s * zr, zr)],
                        sumsr.at[c, pl.ds(s * zr, zr)])

    return k(vals, idx, zrows)


def _sc_counts(idx, npad):
    E = idx.shape[0]
    assert E % EBLK == 0 and npad % 128 == 0, (E, npad)
    rows = E // NW
    nch = rows // CH
    zr = npad // 16
    zrows = jnp.zeros((zr, H), F32)
    ones = jnp.ones((CH, H), F32)

    @functools.partial(
        pl.kernel, mesh=_sc_mesh(),
        out_type=jax.ShapeDtypeStruct((2, npad, H), F32),
        scratch_types=[pltpu.VMEM((CH,), jnp.int32),
                       pltpu.VMEM((CH, H), F32),
                       pltpu.VMEM_SHARED((npad, H), F32)],
    )
    def k(idxr, zrr, onesr, cntr, iv, ob, acc):
        c = lax.axis_index("c")
        s = lax.axis_index("s")
        wid = s * 2 + c
        pltpu.sync_copy(zrr, acc.at[pl.ds(s * zr, zr)])
        pltpu.sync_copy(onesr, ob)
        plsc.subcore_barrier()

        def body(ci, carry):
            base = wid * rows + ci * CH
            pltpu.sync_copy(idxr.at[pl.ds(base, CH)], iv)
            pltpu.sync_copy(ob, acc.at[iv], add=True)
            return carry

        lax.fori_loop(0, nch, body, 0)
        plsc.subcore_barrier()
        pltpu.sync_copy(acc.at[pl.ds(s * zr, zr)],
                        cntr.at[c, pl.ds(s * zr, zr)])

    return k(idx, zrows, ones)


def _pad_to(x, n, value):
    if x.shape[0] == n:
        return x
    pad = [(0, n - x.shape[0])] + [(0, 0)] * (x.ndim - 1)
    return jnp.pad(x, pad, constant_values=value)


def _roundup(e):
    return ((e + EBLK - 1) // EBLK) * EBLK


def kernel(pc2g_edge_idx, pc2g_edge_features, pc2g_node_features,
           g2g_edge_idx, g2g_edge_features, g2pc_edge_idx, g2pc_edge_features,
           params):
    NPC = pc2g_node_features.shape[0]
    NG = _NG
    NGP = ((NG + 1 + 127) // 128) * 128
    NPCP = ((NPC + 1 + 127) // 128) * 128
    gmod = globals()
    gather_pair = gmod["_sc_gather_pair"]
    scatter = gmod["_sc_scatter"]
    counts = gmod["_sc_counts"]

    enc, proc, dec = params["enc"], params["proc"], params["dec"]

    e1 = _ffb(pc2g_edge_features, enc["embed_edge"])
    n = _ffb(pc2g_node_features, enc["embed_node"])
    s1 = pc2g_edge_idx[:, 0]
    r1 = pc2g_edge_idx[:, 1]
    ns, nr = gather_pair(n, s1, n, r1)
    e2 = _edge_mlp(ns, nr, e1, enc["gn_edge"])
    sums = scatter(e2, r1, NGP)
    cnt1 = counts(r1, NGP)
    n_lat = _enc_node(n[:NG], sums[0, :NG], sums[1, :NG],
                      cnt1[0, :NG], cnt1[1, :NG], enc["gn_node"], enc["out"])

    E2 = g2g_edge_idx.shape[0]
    E2P = _roundup(E2)
    pe = _ffb(_pad_to(g2g_edge_features, E2P, 0.0), proc["embed_edge"])
    s2g = _pad_to(g2g_edge_idx[:, 0], E2P, 0)
    r2g = _pad_to(g2g_edge_idx[:, 1], E2P, 0)
    r2s = _pad_to(g2g_edge_idx[:, 1], E2P, NG)
    cnt2 = counts(r2s, NGP)
    for gp in proc["gn"]:
        ns2, nr2 = gather_pair(n_lat, s2g, n_lat, r2g)
        ne, pe = _proc_edge_mlp(ns2, nr2, pe, gp["edge"])
        sums2 = scatter(ne, r2s, NGP)
        n_lat = _proc_node(n_lat, sums2[0, :NG], sums2[1, :NG],
                           cnt2[0, :NG], cnt2[1, :NG], gp["node"])
    n_lat = _ffb(n_lat, proc["out"])

    de = _ffb(g2pc_edge_features, dec["embed_edge"])
    s3 = g2pc_edge_idx[:, 0]
    r3 = g2pc_edge_idx[:, 1]
    ns3, nr3 = gather_pair(n_lat, s3, n, r3)
    de2 = _edge_mlp(ns3, nr3, de, dec["proc_edge"])
    sums3 = scatter(de2, r3, NPCP)
    cnt3 = counts(r3, NPCP)
    out = _dec_node(n, sums3[0, :NPC], sums3[1, :NPC],
                    cnt3[0, :NPC], cnt3[1, :NPC],
                    dec["proc_node"], dec["out"])
    return out
